# Initial kernel scaffold; baseline (speedup 1.0000x reference)
import jax
import jax.numpy as jnp
from jax.experimental import pallas as pl


def kernel(senA, senB, table):
    raise NotImplementedError("write your pallas kernel here")


# speedup vs baseline: 3.1271x; 3.1271x over previous
import functools

import jax
import jax.numpy as jnp
from jax import lax
from jax.experimental import pallas as pl
from jax.experimental.pallas import tpu as pltpu
from jax.experimental.pallas import tpu_sc as plsc

VOCAB = 100000
EMBED_DIM = 128
BATCH = 4096
SEQ = 50

_INFO = plsc.get_sparse_core_info()
NC = _INFO.num_cores
NS = _INFO.num_subcores
NW = NC * NS

ROWS = BATCH * SEQ
ROWS_PER_W = ROWS // NW
CHUNK = 128
NCHUNK = ROWS_PER_W // CHUNK
IDX_ROWS = ROWS_PER_W // CHUNK


def _body(senA_hbm, senB_hbm, table_hbm, outA_hbm, outB_hbm,
          idx_v, rows_v, gsem):
    wid = lax.axis_index("s") * NC + lax.axis_index("c")
    base = wid * ROWS_PER_W

    for sen_hbm, out_hbm in ((senA_hbm, outA_hbm), (senB_hbm, outB_hbm)):
        pltpu.sync_copy(sen_hbm.at[wid], idx_v)

        def chunk(j, _):
            pltpu.async_copy(table_hbm.at[idx_v.at[j]], rows_v, gsem).wait()
            pltpu.sync_copy(rows_v, out_hbm.at[pl.ds(base + j * CHUNK, CHUNK)])
            return _

        lax.fori_loop(0, NCHUNK, chunk, None)


@jax.jit
def _gather_all(senA2d, senB2d, table):
    mesh = plsc.VectorSubcoreMesh(core_axis_name="c", subcore_axis_name="s")
    kern = pl.kernel(
        _body,
        out_type=(
            jax.ShapeDtypeStruct((ROWS, EMBED_DIM), jnp.float32),
            jax.ShapeDtypeStruct((ROWS, EMBED_DIM), jnp.float32),
        ),
        mesh=mesh,
        scratch_types=[
            pltpu.VMEM((IDX_ROWS, CHUNK), jnp.int32),
            pltpu.VMEM((CHUNK, EMBED_DIM), jnp.float32),
            pltpu.SemaphoreType.DMA,
        ],
    )
    return kern(senA2d, senB2d, table)


def kernel(senA, senB, table):
    senA2d = senA.reshape(NW, IDX_ROWS, CHUNK)
    senB2d = senB.reshape(NW, IDX_ROWS, CHUNK)
    outA, outB = _gather_all(senA2d, senB2d, table)
    return (outA.reshape(BATCH, SEQ, EMBED_DIM),
            outB.reshape(BATCH, SEQ, EMBED_DIM))


# speedup vs baseline: 3.5757x; 1.1435x over previous
import functools

import jax
import jax.numpy as jnp
from jax import lax
from jax.experimental import pallas as pl
from jax.experimental.pallas import tpu as pltpu
from jax.experimental.pallas import tpu_sc as plsc

VOCAB = 100000
EMBED_DIM = 128
BATCH = 4096
SEQ = 50

_INFO = plsc.get_sparse_core_info()
NC = _INFO.num_cores
NS = _INFO.num_subcores
NW = NC * NS

ROWS = BATCH * SEQ
ROWS_PER_W = ROWS // NW
CHUNK = 128
NCHUNK = ROWS_PER_W // CHUNK
IDX_ROWS = ROWS_PER_W // CHUNK


NBUF = 5


def _body(senA_hbm, senB_hbm, table_hbm, outA_hbm, outB_hbm,
          idx_v, rows_bufs, gsems, ssems):
    wid = lax.axis_index("s") * NC + lax.axis_index("c")
    base = wid * ROWS_PER_W

    for sen_hbm, out_hbm in ((senA_hbm, outA_hbm), (senB_hbm, outB_hbm)):
        pltpu.sync_copy(sen_hbm.at[wid], idx_v)

        for b in range(NBUF):
            pltpu.async_copy(table_hbm.at[idx_v.at[b]], rows_bufs[b], gsems[b])

        def round_(j0, _):
            for b in range(NBUF):
                j = j0 + b
                pltpu.make_async_copy(table_hbm.at[idx_v.at[j]], rows_bufs[b],
                                      gsems[b]).wait()
                st = pltpu.async_copy(
                    rows_bufs[b], out_hbm.at[pl.ds(base + j * CHUNK, CHUNK)],
                    ssems[b])

                @pl.when(j < NCHUNK - NBUF)
                def _():
                    st.wait()
                    pltpu.async_copy(table_hbm.at[idx_v.at[j + NBUF]],
                                     rows_bufs[b], gsems[b])
            return _

        lax.fori_loop(0, NCHUNK // NBUF, lambda i, c: round_(i * NBUF, c),
                      None)

        for b in range(NBUF):
            pltpu.make_async_copy(
                rows_bufs[b],
                out_hbm.at[pl.ds(base + (NCHUNK - NBUF + b) * CHUNK, CHUNK)],
                ssems[b]).wait()


@jax.jit
def _gather_all(senA2d, senB2d, table):
    mesh = plsc.VectorSubcoreMesh(core_axis_name="c", subcore_axis_name="s")
    kern = pl.kernel(
        _body,
        out_type=(
            jax.ShapeDtypeStruct((ROWS, EMBED_DIM), jnp.float32),
            jax.ShapeDtypeStruct((ROWS, EMBED_DIM), jnp.float32),
        ),
        mesh=mesh,
        scratch_types=[
            pltpu.VMEM((IDX_ROWS, CHUNK), jnp.int32),
            [pltpu.VMEM((CHUNK, EMBED_DIM), jnp.float32)
             for _ in range(NBUF)],
            [pltpu.SemaphoreType.DMA for _ in range(NBUF)],
            [pltpu.SemaphoreType.DMA for _ in range(NBUF)],
        ],
    )
    return kern(senA2d, senB2d, table)


def kernel(senA, senB, table):
    senA2d = senA.reshape(NW, IDX_ROWS, CHUNK)
    senB2d = senB.reshape(NW, IDX_ROWS, CHUNK)
    outA, outB = _gather_all(senA2d, senB2d, table)
    return (outA.reshape(BATCH, SEQ, EMBED_DIM),
            outB.reshape(BATCH, SEQ, EMBED_DIM))


# speedup vs baseline: 6.0504x; 1.6921x over previous
import functools

import jax
import jax.numpy as jnp
from jax import lax
from jax.experimental import pallas as pl
from jax.experimental.pallas import tpu as pltpu
from jax.experimental.pallas import tpu_sc as plsc

VOCAB = 100000
EMBED_DIM = 128
BATCH = 4096
SEQ = 50

_INFO = plsc.get_sparse_core_info()
NC = _INFO.num_cores
NS = _INFO.num_subcores
NW = NC * NS

SAMPLES_PER_W = BATCH // NW
PAIR = 2
CHUNK_IDX = PAIR * SEQ
NCHUNK = SAMPLES_PER_W // PAIR
NBUF = 4


def _body(senA_hbm, senB_hbm, table_hbm, outA_hbm, outB_hbm,
          idx_v, rows_bufs, gsems, ssems):
    wid = lax.axis_index("s") * NC + lax.axis_index("c")
    sbase = wid * SAMPLES_PER_W

    def store_pair(b, j):
        sts = []
        for k in range(PAIR):
            sts.append(pltpu.async_copy(
                rows_bufs[b].at[pl.ds(k * SEQ, SEQ)],
                out_hbm.at[sbase + j * PAIR + k], ssems[b]))
        return sts

    for sen_hbm, out_hbm in ((senA_hbm, outA_hbm), (senB_hbm, outB_hbm)):
        pltpu.sync_copy(sen_hbm.at[wid], idx_v)

        for b in range(NBUF):
            pltpu.async_copy(table_hbm.at[idx_v.at[b]], rows_bufs[b], gsems[b])

        def round_(j0, _):
            for b in range(NBUF):
                j = j0 + b
                pltpu.make_async_copy(table_hbm.at[idx_v.at[j]], rows_bufs[b],
                                      gsems[b]).wait()
                sts = store_pair(b, j)

                @pl.when(j < NCHUNK - NBUF)
                def _():
                    for st in sts:
                        st.wait()
                    pltpu.async_copy(table_hbm.at[idx_v.at[j + NBUF]],
                                     rows_bufs[b], gsems[b])
            return _

        lax.fori_loop(0, NCHUNK // NBUF, lambda i, c: round_(i * NBUF, c),
                      None)

        for b in range(NBUF):
            j = NCHUNK - NBUF + b
            for k in range(PAIR):
                pltpu.make_async_copy(
                    rows_bufs[b].at[pl.ds(k * SEQ, SEQ)],
                    out_hbm.at[sbase + j * PAIR + k], ssems[b]).wait()


@jax.jit
def _gather_all(senA3, senB3, table):
    mesh = plsc.VectorSubcoreMesh(core_axis_name="c", subcore_axis_name="s")
    kern = pl.kernel(
        _body,
        out_type=(
            jax.ShapeDtypeStruct((BATCH, SEQ, EMBED_DIM), jnp.float32),
            jax.ShapeDtypeStruct((BATCH, SEQ, EMBED_DIM), jnp.float32),
        ),
        mesh=mesh,
        scratch_types=[
            pltpu.VMEM((NCHUNK, CHUNK_IDX), jnp.int32),
            [pltpu.VMEM((CHUNK_IDX, EMBED_DIM), jnp.float32)
             for _ in range(NBUF)],
            [pltpu.SemaphoreType.DMA for _ in range(NBUF)],
            [pltpu.SemaphoreType.DMA for _ in range(NBUF)],
        ],
    )
    return kern(senA3, senB3, table)


def kernel(senA, senB, table):
    senA3 = senA.reshape(NW, NCHUNK, CHUNK_IDX)
    senB3 = senB.reshape(NW, NCHUNK, CHUNK_IDX)
    return _gather_all(senA3, senB3, table)


# speedup vs baseline: 11.1913x; 1.8497x over previous
import functools

import jax
import jax.numpy as jnp
from jax import lax
from jax.experimental import pallas as pl
from jax.experimental.pallas import tpu as pltpu
from jax.experimental.pallas import tpu_sc as plsc

VOCAB = 100000
EMBED_DIM = 128
BATCH = 4096
SEQ = 50

_INFO = plsc.get_sparse_core_info()
NC = _INFO.num_cores
NS = _INFO.num_subcores
NW = NC * NS

SAMPLES_PER_W = BATCH // NW
NCHUNK = SEQ
NBUF = 5
LAG = 2


def _body(senA_hbm, senB_hbm, table_hbm, outA_hbm, outB_hbm,
          idxA_v, idxB_v, rows_bufs, gsems, ssems):
    wid = lax.axis_index("s") * NC + lax.axis_index("c")
    sbase = wid * SAMPLES_PER_W

    pltpu.sync_copy(senA_hbm.at[wid], idxA_v)
    pltpu.sync_copy(senB_hbm.at[wid], idxB_v)

    def issue_gather(c, b):
        @pl.when(c < NCHUNK)
        def _():
            pltpu.async_copy(table_hbm.at[idxA_v.at[c]], rows_bufs[b],
                             gsems[b])

        @pl.when(c >= NCHUNK)
        def _():
            pltpu.async_copy(table_hbm.at[idxB_v.at[c - NCHUNK]], rows_bufs[b],
                             gsems[b])

    def issue_store(c, b):
        @pl.when(c < NCHUNK)
        def _():
            pltpu.async_copy(rows_bufs[b],
                             outA_hbm.at[c, pl.ds(sbase, SAMPLES_PER_W)],
                             ssems[b])

        @pl.when(c >= NCHUNK)
        def _():
            pltpu.async_copy(rows_bufs[b],
                             outB_hbm.at[c - NCHUNK,
                                         pl.ds(sbase, SAMPLES_PER_W)],
                             ssems[b])

    for b in range(NBUF):
        issue_gather(b, b)

    TOT = 2 * NCHUNK

    def round_(c0, _):
        for b in range(NBUF):
            c = c0 + b
            pltpu.make_async_copy(table_hbm.at[idxA_v.at[0]], rows_bufs[b],
                                  gsems[b]).wait()
            issue_store(c, b)

            @pl.when(c < TOT - NBUF)
            def _():
                pltpu.make_async_copy(
                    rows_bufs[b], outA_hbm.at[0, pl.ds(sbase, SAMPLES_PER_W)],
                    ssems[b]).wait()
                issue_gather(c + NBUF, b)
        return _

    lax.fori_loop(0, TOT // NBUF, lambda i, c: round_(i * NBUF, c), None)

    for b in range(NBUF):
        pltpu.make_async_copy(
            rows_bufs[b], outA_hbm.at[0, pl.ds(sbase, SAMPLES_PER_W)],
            ssems[b]).wait()


@jax.jit
def _gather_all(senA3, senB3, table):
    mesh = plsc.VectorSubcoreMesh(core_axis_name="c", subcore_axis_name="s")
    kern = pl.kernel(
        _body,
        out_type=(
            jax.ShapeDtypeStruct((SEQ, BATCH, EMBED_DIM), jnp.float32),
            jax.ShapeDtypeStruct((SEQ, BATCH, EMBED_DIM), jnp.float32),
        ),
        mesh=mesh,
        scratch_types=[
            pltpu.VMEM((NCHUNK, SAMPLES_PER_W), jnp.int32),
            pltpu.VMEM((NCHUNK, SAMPLES_PER_W), jnp.int32),
            [pltpu.VMEM((SAMPLES_PER_W, EMBED_DIM), jnp.float32)
             for _ in range(NBUF)],
            [pltpu.SemaphoreType.DMA for _ in range(NBUF)],
            [pltpu.SemaphoreType.DMA for _ in range(NBUF)],
        ],
    )
    return kern(senA3, senB3, table)


def kernel(senA, senB, table):
    senA3 = senA.T.reshape(SEQ, NW, SAMPLES_PER_W).transpose(1, 0, 2)
    senB3 = senB.T.reshape(SEQ, NW, SAMPLES_PER_W).transpose(1, 0, 2)
    outA3, outB3 = _gather_all(senA3, senB3, table)
    return outA3.transpose(1, 0, 2), outB3.transpose(1, 0, 2)


# speedup vs baseline: 11.2762x; 1.0076x over previous
import functools

import jax
import jax.numpy as jnp
from jax import lax
from jax.experimental import pallas as pl
from jax.experimental.pallas import tpu as pltpu
from jax.experimental.pallas import tpu_sc as plsc

VOCAB = 100000
EMBED_DIM = 128
BATCH = 4096
SEQ = 50

_INFO = plsc.get_sparse_core_info()
NC = _INFO.num_cores
NS = _INFO.num_subcores
NW = NC * NS

SAMPLES_PER_W = BATCH // NW
NCHUNK = SEQ
NBUF = 5
LAG = 2


def _body(senA_hbm, senB_hbm, table_hbm, outA_hbm, outB_hbm,
          idx_v, rows_bufs, gsems, ssems):
    wid = lax.axis_index("s") * NC + lax.axis_index("c")
    sbase = wid * SAMPLES_PER_W

    for sen_hbm, out_hbm in ((senA_hbm, outA_hbm), (senB_hbm, outB_hbm)):
        pltpu.sync_copy(sen_hbm.at[wid], idx_v)

        for b in range(NBUF):
            pltpu.async_copy(table_hbm.at[idx_v.at[b]], rows_bufs[b], gsems[b])

        def round_(t0, _):
            for b in range(NBUF):
                t = t0 + b
                pltpu.make_async_copy(table_hbm.at[idx_v.at[t]], rows_bufs[b],
                                      gsems[b]).wait()
                pltpu.async_copy(
                    rows_bufs[b], out_hbm.at[t, pl.ds(sbase, SAMPLES_PER_W)],
                    ssems[b])

                b2 = (b - LAG) % NBUF
                t_new = t - LAG + NBUF

                @pl.when(jnp.logical_and(t >= LAG, t_new < NCHUNK))
                def _():
                    pltpu.make_async_copy(
                        rows_bufs[b2],
                        out_hbm.at[t - LAG, pl.ds(sbase, SAMPLES_PER_W)],
                        ssems[b2]).wait()
                    pltpu.async_copy(table_hbm.at[idx_v.at[t_new]],
                                     rows_bufs[b2], gsems[b2])
            return _

        lax.fori_loop(0, NCHUNK // NBUF, lambda i, c: round_(i * NBUF, c),
                      None)

        for b in range(NBUF):
            t = NCHUNK - NBUF + b
            pltpu.make_async_copy(
                rows_bufs[b], out_hbm.at[t, pl.ds(sbase, SAMPLES_PER_W)],
                ssems[b]).wait()


@jax.jit
def _gather_all(senA3, senB3, table):
    mesh = plsc.VectorSubcoreMesh(core_axis_name="c", subcore_axis_name="s")
    kern = pl.kernel(
        _body,
        out_type=(
            jax.ShapeDtypeStruct((SEQ, BATCH, EMBED_DIM), jnp.float32),
            jax.ShapeDtypeStruct((SEQ, BATCH, EMBED_DIM), jnp.float32),
        ),
        mesh=mesh,
        scratch_types=[
            pltpu.VMEM((NCHUNK, SAMPLES_PER_W), jnp.int32),
            [pltpu.VMEM((SAMPLES_PER_W, EMBED_DIM), jnp.float32)
             for _ in range(NBUF)],
            [pltpu.SemaphoreType.DMA for _ in range(NBUF)],
            [pltpu.SemaphoreType.DMA for _ in range(NBUF)],
        ],
    )
    return kern(senA3, senB3, table)


def kernel(senA, senB, table):
    senA3 = senA.T.reshape(SEQ, NW, SAMPLES_PER_W).transpose(1, 0, 2)
    senB3 = senB.T.reshape(SEQ, NW, SAMPLES_PER_W).transpose(1, 0, 2)
    outA3, outB3 = _gather_all(senA3, senB3, table)
    return outA3.transpose(1, 0, 2), outB3.transpose(1, 0, 2)
